# Initial kernel scaffold; baseline (speedup 1.0000x reference)
import jax
import jax.numpy as jnp
from jax.experimental import pallas as pl


def kernel(x, edge_index, W1, att_src1, att_dst1, b1, W2, att_src2, att_dst2, b2):
    raise NotImplementedError("write your pallas kernel here")


# speedup vs baseline: 89.4812x; 89.4812x over previous
import functools

import jax
import jax.numpy as jnp
from jax import lax
from jax.experimental import pallas as pl
from jax.experimental.pallas import tpu as pltpu
from jax.experimental.pallas import tpu_sc as plsc

N = 10000
E = 320000
D = 128
H = 8
C = 8
HD = H * C
WD = 128
NEG = 0.2

NC = 2
NS = 16
NW = NC * NS
ET = E // NW
CH = 80
NCHUNK = ET // CH
RPT = 624

_MM = dict(preferred_element_type=jnp.float32, precision=lax.Precision.HIGHEST)


def _start_body(x_ref, w_ref, p_ref, xpaug_ref):
    xp = jnp.dot(x_ref[...], w_ref[...], **_MM)
    xpaug_ref[...] = jnp.dot(xp, p_ref[...], **_MM)


def _bridge_body(accA_ref, accB_ref, xpaug_ref, b_ref, w_ref,
                 p_ref, r8_ref, xpaug_o_ref):
    accA = accA_ref[...]
    accB = accB_ref[...]
    xpaug = xpaug_ref[...]
    num = accA[:, :64] + accB[:, :64]
    den = accA[:, 64:72] + accB[:, 64:72]
    alpha = xpaug[:, 64:72] + xpaug[:, 80:88]
    exs = jnp.exp(jnp.where(alpha >= 0, alpha, NEG * alpha))
    r8 = r8_ref[...]
    num = num + xpaug[:, :64] * jnp.dot(exs, r8, **_MM)
    den = den + exs
    h = num / jnp.dot(den, r8, **_MM) + b_ref[...]
    xpaug_o_ref[...] = jnp.dot(jnp.dot(h, w_ref[...], **_MM), p_ref[...], **_MM)


def _finish_body(accA_ref, accB_ref, xpaug_ref, b_ref, r8_ref, out_ref):
    accA = accA_ref[...]
    accB = accB_ref[...]
    xpaug = xpaug_ref[...]
    num = accA[:, :64] + accB[:, :64]
    den = accA[:, 64:72] + accB[:, 64:72]
    alpha = xpaug[:, 64:72] + xpaug[:, 80:88]
    exs = jnp.exp(jnp.where(alpha >= 0, alpha, NEG * alpha))
    r8 = r8_ref[...]
    num = num + xpaug[:, :64] * jnp.dot(exs, r8, **_MM)
    den = den + exs
    h = num / jnp.dot(den, r8, **_MM) + b_ref[...]
    m = jnp.max(h, axis=1, keepdims=True)
    lse = m + jnp.log(jnp.sum(jnp.exp(h - m), axis=1, keepdims=True))
    out_ref[...] = h - lse


_BLK = 1000
_GRID = N // _BLK


def _row_spec(width):
    return pl.BlockSpec((_BLK, width), lambda i: (i, 0))


def _full_spec(shape):
    return pl.BlockSpec(shape, lambda i: tuple(0 for _ in shape))


def _tc_start(x, w, p):
    return pl.pallas_call(
        _start_body,
        grid=(_GRID,),
        in_specs=[_row_spec(D), _full_spec((D, HD)), _full_spec((HD, WD))],
        out_specs=_row_spec(WD),
        out_shape=jax.ShapeDtypeStruct((N, WD), jnp.float32),
    )(x, w, p)


def _tc_bridge(accA, accB, xpaug, b, w, p, r8):
    return pl.pallas_call(
        _bridge_body,
        grid=(_GRID,),
        in_specs=[_row_spec(WD), _row_spec(WD), _row_spec(WD),
                  _full_spec((1, HD)), _full_spec((HD, HD)),
                  _full_spec((HD, WD)), _full_spec((H, HD))],
        out_specs=_row_spec(WD),
        out_shape=jax.ShapeDtypeStruct((N, WD), jnp.float32),
    )(accA, accB, xpaug, b, w, p, r8)


def _tc_finish(accA, accB, xpaug, b, r8):
    return pl.pallas_call(
        _finish_body,
        grid=(_GRID,),
        in_specs=[_row_spec(WD), _row_spec(WD), _row_spec(WD),
                  _full_spec((1, HD)), _full_spec((H, HD))],
        out_specs=_row_spec(HD),
        out_shape=jax.ShapeDtypeStruct((N, HD), jnp.float32),
    )(accA, accB, xpaug, b, r8)


def _sc_edge_body(src_hbm, dst_hbm, xpaug_hbm, zeros_hbm, out_hbm,
                  acc, srcv, dstv, rowsS, rowsD, outv, sem1, sem2):
    c = lax.axis_index("c")
    s = lax.axis_index("s")
    wid = c * NS + s

    pltpu.sync_copy(zeros_hbm.at[pl.ds(s * RPT, RPT)],
                    acc.at[pl.ds(s * RPT, RPT)])

    @pl.when(s == NS - 1)
    def _():
        pltpu.sync_copy(zeros_hbm.at[pl.ds(NS * RPT, N - NS * RPT)],
                        acc.at[pl.ds(NS * RPT, N - NS * RPT)])

    zv = jnp.zeros((16,), jnp.float32)
    for r in range(CH):
        for k3 in range(3):
            outv[r, pl.ds(80 + 16 * k3, 16)] = zv
    plsc.subcore_barrier()

    iota = lax.iota(jnp.int32, 16)
    lane_lt8 = iota < 8
    base = wid * ET

    def chunk_body(i, carry):
        off = base + i * CH
        pltpu.sync_copy(src_hbm.at[pl.ds(off, CH)], srcv)
        pltpu.sync_copy(dst_hbm.at[pl.ds(off, CH)], dstv)
        cp1 = pltpu.async_copy(xpaug_hbm.at[srcv], rowsS, sem1)
        cp2 = pltpu.async_copy(xpaug_hbm.at[dstv], rowsD, sem2)
        cp1.wait()
        cp2.wait()

        def edge_body(e, cc):
            va = rowsS[e, pl.ds(64, 16)]
            vd = rowsD[e, pl.ds(80, 16)]
            al = va + vd
            ex = jnp.exp(jnp.where(al >= 0, al, NEG * al))
            ex = jnp.where(lane_lt8, ex, 0.0)
            outv[e, pl.ds(64, 16)] = ex
            es = [jnp.broadcast_to(ex[j], (16,)) for j in range(8)]
            for k in range(4):
                v = rowsS[e, pl.ds(k * 16, 16)]
                exb = jnp.where(lane_lt8, es[2 * k], es[2 * k + 1])
                outv[e, pl.ds(k * 16, 16)] = v * exb
            return cc

        lax.fori_loop(0, CH, edge_body, 0)
        pltpu.sync_copy(outv, acc.at[dstv], add=True)
        return carry

    lax.fori_loop(0, NCHUNK, chunk_body, 0)
    plsc.subcore_barrier()
    pltpu.sync_copy(acc.at[pl.ds(s * RPT, RPT)],
                    out_hbm.at[c, pl.ds(s * RPT, RPT)])

    @pl.when(s == NS - 1)
    def _():
        pltpu.sync_copy(acc.at[pl.ds(NS * RPT, N - NS * RPT)],
                        out_hbm.at[c, pl.ds(NS * RPT, N - NS * RPT)])


_sc_edge = functools.partial(
    pl.kernel,
    out_type=jax.ShapeDtypeStruct((NC, N, WD), jnp.float32),
    mesh=plsc.VectorSubcoreMesh(core_axis_name="c", subcore_axis_name="s"),
    scratch_types=[
        pltpu.VMEM_SHARED((N, WD), jnp.float32),
        pltpu.VMEM((CH,), jnp.int32),
        pltpu.VMEM((CH,), jnp.int32),
        pltpu.VMEM((CH, WD), jnp.float32),
        pltpu.VMEM((CH, WD), jnp.float32),
        pltpu.VMEM((CH, WD), jnp.float32),
        pltpu.SemaphoreType.DMA,
        pltpu.SemaphoreType.DMA,
    ],
)(_sc_edge_body)


def _build_p(att_src, att_dst):
    p = jnp.zeros((HD, WD), jnp.float32)
    p = p.at[:, :HD].set(jnp.eye(HD, dtype=jnp.float32))
    rows = jnp.arange(HD)
    heads = rows // C
    p = p.at[rows, 64 + heads].set(att_src.reshape(-1))
    p = p.at[rows, 80 + heads].set(att_dst.reshape(-1))
    return p


def kernel(x, edge_index, W1, att_src1, att_dst1, b1, W2, att_src2, att_dst2, b2):
    src = edge_index[0].astype(jnp.int32)
    dst = edge_index[1].astype(jnp.int32)
    p1 = _build_p(att_src1, att_dst1)
    p2 = _build_p(att_src2, att_dst2)
    r8 = jnp.repeat(jnp.eye(H, dtype=jnp.float32), C, axis=1)
    zeros = jnp.zeros((N, WD), jnp.float32)
    b1r = b1.reshape(1, HD)
    b2r = b2.reshape(1, HD)

    xpaug1 = _tc_start(x, W1, p1)
    acc1 = _sc_edge(src, dst, xpaug1, zeros)
    xpaug2 = _tc_bridge(acc1[0], acc1[1], xpaug1, b1r, W2, p2, r8)
    acc2 = _sc_edge(src, dst, xpaug2, zeros)
    xpaug3 = _tc_bridge(acc2[0], acc2[1], xpaug2, b2r, W2, p2, r8)
    acc3 = _sc_edge(src, dst, xpaug3, zeros)
    return _tc_finish(acc3[0], acc3[1], xpaug3, b2r, r8)


# speedup vs baseline: 107.7678x; 1.2044x over previous
import functools

import jax
import jax.numpy as jnp
from jax import lax
from jax.experimental import pallas as pl
from jax.experimental.pallas import tpu as pltpu
from jax.experimental.pallas import tpu_sc as plsc

N = 10000
E = 320000
D = 128
H = 8
C = 8
HD = H * C
WD = 128
NEG = 0.2

NC = 2
NS = 16
NW = NC * NS
ET = E // NW
CH = 80
NCHUNK = ET // CH
NPAIR = NCHUNK // 2
RPT = 624

_MM = dict(preferred_element_type=jnp.float32, precision=lax.Precision.HIGHEST)


def _start_body(x_ref, w_ref, p_ref, xpaug_ref):
    xp = jnp.dot(x_ref[...], w_ref[...], **_MM)
    xpaug_ref[...] = jnp.dot(xp, p_ref[...], **_MM)


def _bridge_body(accA_ref, accB_ref, xpaug_ref, b_ref, w_ref,
                 p_ref, r8_ref, xpaug_o_ref):
    accA = accA_ref[...]
    accB = accB_ref[...]
    xpaug = xpaug_ref[...]
    num = accA[:, :64] + accB[:, :64]
    den = accA[:, 64:72] + accB[:, 64:72]
    alpha = xpaug[:, 64:72] + xpaug[:, 80:88]
    exs = jnp.exp(jnp.where(alpha >= 0, alpha, NEG * alpha))
    r8 = r8_ref[...]
    num = num + xpaug[:, :64] * jnp.dot(exs, r8, **_MM)
    den = den + exs
    h = num / jnp.dot(den, r8, **_MM) + b_ref[...]
    xpaug_o_ref[...] = jnp.dot(jnp.dot(h, w_ref[...], **_MM), p_ref[...], **_MM)


def _finish_body(accA_ref, accB_ref, xpaug_ref, b_ref, r8_ref, out_ref):
    accA = accA_ref[...]
    accB = accB_ref[...]
    xpaug = xpaug_ref[...]
    num = accA[:, :64] + accB[:, :64]
    den = accA[:, 64:72] + accB[:, 64:72]
    alpha = xpaug[:, 64:72] + xpaug[:, 80:88]
    exs = jnp.exp(jnp.where(alpha >= 0, alpha, NEG * alpha))
    r8 = r8_ref[...]
    num = num + xpaug[:, :64] * jnp.dot(exs, r8, **_MM)
    den = den + exs
    h = num / jnp.dot(den, r8, **_MM) + b_ref[...]
    m = jnp.max(h, axis=1, keepdims=True)
    lse = m + jnp.log(jnp.sum(jnp.exp(h - m), axis=1, keepdims=True))
    out_ref[...] = h - lse


_BLK = 1000
_GRID = N // _BLK


def _row_spec(width):
    return pl.BlockSpec((_BLK, width), lambda i: (i, 0))


def _full_spec(shape):
    return pl.BlockSpec(shape, lambda i: tuple(0 for _ in shape))


def _tc_start(x, w, p):
    return pl.pallas_call(
        _start_body,
        grid=(_GRID,),
        in_specs=[_row_spec(D), _full_spec((D, HD)), _full_spec((HD, WD))],
        out_specs=_row_spec(WD),
        out_shape=jax.ShapeDtypeStruct((N, WD), jnp.float32),
    )(x, w, p)


def _tc_bridge(accA, accB, xpaug, b, w, p, r8):
    return pl.pallas_call(
        _bridge_body,
        grid=(_GRID,),
        in_specs=[_row_spec(WD), _row_spec(WD), _row_spec(WD),
                  _full_spec((1, HD)), _full_spec((HD, HD)),
                  _full_spec((HD, WD)), _full_spec((H, HD))],
        out_specs=_row_spec(WD),
        out_shape=jax.ShapeDtypeStruct((N, WD), jnp.float32),
    )(accA, accB, xpaug, b, w, p, r8)


def _tc_finish(accA, accB, xpaug, b, r8):
    return pl.pallas_call(
        _finish_body,
        grid=(_GRID,),
        in_specs=[_row_spec(WD), _row_spec(WD), _row_spec(WD),
                  _full_spec((1, HD)), _full_spec((H, HD))],
        out_specs=_row_spec(HD),
        out_shape=jax.ShapeDtypeStruct((N, HD), jnp.float32),
    )(accA, accB, xpaug, b, r8)


def _sc_edge_body(src_hbm, dst_hbm, xpaug_hbm, zeros_hbm, out_hbm,
                  acc, srcvA, dstvA, rowsSA, rowsDA,
                  srcvB, dstvB, rowsSB, rowsDB,
                  semSA, semDA, semSB, semDB):
    c = lax.axis_index("c")
    s = lax.axis_index("s")
    wid = c * NS + s

    pltpu.sync_copy(zeros_hbm.at[pl.ds(s * RPT, RPT)],
                    acc.at[pl.ds(s * RPT, RPT)])

    @pl.when(s == NS - 1)
    def _():
        pltpu.sync_copy(zeros_hbm.at[pl.ds(NS * RPT, N - NS * RPT)],
                        acc.at[pl.ds(NS * RPT, N - NS * RPT)])

    plsc.subcore_barrier()

    iota = lax.iota(jnp.int32, 16)
    lane_lt8 = iota < 8
    base = wid * ET

    def compute_scatter(rowsS, rowsD, dstv):
        def edge_body(e, cc):
            va = rowsS[e, pl.ds(64, 16)]
            vd = rowsD[e, pl.ds(80, 16)]
            al = va + vd
            ex = jnp.exp(jnp.where(al >= 0, al, NEG * al))
            es = [jnp.broadcast_to(ex[j], (16,)) for j in range(8)]
            for k in range(4):
                v = rowsS[e, pl.ds(k * 16, 16)]
                exb = jnp.where(lane_lt8, es[2 * k], es[2 * k + 1])
                rowsS[e, pl.ds(k * 16, 16)] = v * exb
            rowsS[e, pl.ds(64, 16)] = ex
            return cc

        lax.fori_loop(0, CH, edge_body, 0)
        pltpu.sync_copy(rowsS, acc.at[dstv], add=True)

    def issue(i, srcv, dstv, rowsS, rowsD, semS, semD):
        off = base + i * CH
        pltpu.sync_copy(src_hbm.at[pl.ds(off, CH)], srcv)
        pltpu.sync_copy(dst_hbm.at[pl.ds(off, CH)], dstv)
        pltpu.make_async_copy(xpaug_hbm.at[srcv], rowsS, semS).start()
        pltpu.make_async_copy(xpaug_hbm.at[dstv], rowsD, semD).start()

    def waitA():
        pltpu.make_async_copy(xpaug_hbm.at[srcvA], rowsSA, semSA).wait()
        pltpu.make_async_copy(xpaug_hbm.at[dstvA], rowsDA, semDA).wait()

    def waitB():
        pltpu.make_async_copy(xpaug_hbm.at[srcvB], rowsSB, semSB).wait()
        pltpu.make_async_copy(xpaug_hbm.at[dstvB], rowsDB, semDB).wait()

    issue(0, srcvA, dstvA, rowsSA, rowsDA, semSA, semDA)

    def pair_body(j, carry):
        issue(2 * j + 1, srcvB, dstvB, rowsSB, rowsDB, semSB, semDB)
        waitA()
        compute_scatter(rowsSA, rowsDA, dstvA)
        issue(2 * j + 2, srcvA, dstvA, rowsSA, rowsDA, semSA, semDA)
        waitB()
        compute_scatter(rowsSB, rowsDB, dstvB)
        return carry

    lax.fori_loop(0, NPAIR, pair_body, 0)
    waitA()
    compute_scatter(rowsSA, rowsDA, dstvA)

    plsc.subcore_barrier()
    pltpu.sync_copy(acc.at[pl.ds(s * RPT, RPT)],
                    out_hbm.at[c, pl.ds(s * RPT, RPT)])

    @pl.when(s == NS - 1)
    def _():
        pltpu.sync_copy(acc.at[pl.ds(NS * RPT, N - NS * RPT)],
                        out_hbm.at[c, pl.ds(NS * RPT, N - NS * RPT)])


_sc_edge = functools.partial(
    pl.kernel,
    out_type=jax.ShapeDtypeStruct((NC, N, WD), jnp.float32),
    mesh=plsc.VectorSubcoreMesh(core_axis_name="c", subcore_axis_name="s"),
    scratch_types=[
        pltpu.VMEM_SHARED((N, WD), jnp.float32),
        pltpu.VMEM((CH,), jnp.int32),
        pltpu.VMEM((CH,), jnp.int32),
        pltpu.VMEM((CH, WD), jnp.float32),
        pltpu.VMEM((CH, WD), jnp.float32),
        pltpu.VMEM((CH,), jnp.int32),
        pltpu.VMEM((CH,), jnp.int32),
        pltpu.VMEM((CH, WD), jnp.float32),
        pltpu.VMEM((CH, WD), jnp.float32),
        pltpu.SemaphoreType.DMA,
        pltpu.SemaphoreType.DMA,
        pltpu.SemaphoreType.DMA,
        pltpu.SemaphoreType.DMA,
    ],
)(_sc_edge_body)


def _build_p(att_src, att_dst):
    p = jnp.zeros((HD, WD), jnp.float32)
    p = p.at[:, :HD].set(jnp.eye(HD, dtype=jnp.float32))
    rows = jnp.arange(HD)
    heads = rows // C
    p = p.at[rows, 64 + heads].set(att_src.reshape(-1))
    p = p.at[rows, 80 + heads].set(att_dst.reshape(-1))
    return p


def kernel(x, edge_index, W1, att_src1, att_dst1, b1, W2, att_src2, att_dst2, b2):
    src = edge_index[0].astype(jnp.int32)
    dst = edge_index[1].astype(jnp.int32)
    p1 = _build_p(att_src1, att_dst1)
    p2 = _build_p(att_src2, att_dst2)
    r8 = jnp.repeat(jnp.eye(H, dtype=jnp.float32), C, axis=1)
    zeros = jnp.zeros((N, WD), jnp.float32)
    b1r = b1.reshape(1, HD)
    b2r = b2.reshape(1, HD)

    xpaug1 = _tc_start(x, W1, p1)
    acc1 = _sc_edge(src, dst, xpaug1, zeros)
    xpaug2 = _tc_bridge(acc1[0], acc1[1], xpaug1, b1r, W2, p2, r8)
    acc2 = _sc_edge(src, dst, xpaug2, zeros)
    xpaug3 = _tc_bridge(acc2[0], acc2[1], xpaug2, b2r, W2, p2, r8)
    acc3 = _sc_edge(src, dst, xpaug3, zeros)
    return _tc_finish(acc3[0], acc3[1], xpaug3, b2r, r8)
